# baseline (speedup 1.0000x reference)
import jax
import jax.numpy as jnp
from jax import lax
from jax.experimental import pallas as pl
from jax.experimental.pallas import tpu as pltpu
from jax.experimental.pallas import tpu_sc as plsc

B = 16384
D_MODEL = 64
STRIDE1 = 1000.0

NC = 2
NS = 16
L = 16
NW = NC * NS
B_PER_W = B // NW
GROUPS = B_PER_W // L
CHUNK = 64
N_CHUNKS = B_PER_W // CHUNK


def _sc_body(c0_hbm, c1_hbm, table_hbm, out_hbm,
             c0_v, c1_v, iq_v, rows_v, sems):
    wid = lax.axis_index("s") * NC + lax.axis_index("c")
    base = wid * B_PER_W
    base2 = wid * (B_PER_W // 2)

    pltpu.sync_copy(c0_hbm.at[pl.ds(base, B_PER_W)], c0_v)
    pltpu.sync_copy(c1_hbm.at[pl.ds(base, B_PER_W)], c1_v)

    for g in range(GROUPS):
        v0 = c0_v[pl.ds(g * L, L)]
        v1 = c1_v[pl.ds(g * L, L)]
        iq_v[pl.ds(g * L, L)] = (v0 + STRIDE1 * v1).astype(jnp.int32)

    def fire(g, _):
        vec = iq_v[pl.ds(g * L, L)]
        for j in range(L):
            pltpu.async_copy(
                table_hbm.at[vec[j]],
                rows_v.at[g * (L // 2) + j // 2, pl.ds((j % 2) * D_MODEL, D_MODEL)],
                sems.at[j % 4],
            )
        return 0

    lax.fori_loop(0, GROUPS, fire, 0)
    for k in range(4):
        pltpu.make_async_copy(
            out_hbm.at[pl.ds(base2 + k * (B_PER_W // 8), B_PER_W // 8)],
            rows_v.at[pl.ds(k * (B_PER_W // 8), B_PER_W // 8)],
            sems.at[k],
        ).wait()

    pltpu.sync_copy(rows_v, out_hbm.at[pl.ds(base2, B_PER_W // 2)])


@jax.jit
def kernel(x_entity0, embeddings):
    mesh = plsc.VectorSubcoreMesh(core_axis_name="c", subcore_axis_name="s")
    run = pl.kernel(
        _sc_body,
        out_type=jax.ShapeDtypeStruct((B // 2, 2 * D_MODEL), jnp.float32),
        mesh=mesh,
        scratch_types=[
            pltpu.VMEM((B_PER_W,), jnp.float32),
            pltpu.VMEM((B_PER_W,), jnp.float32),
            pltpu.VMEM((B_PER_W + L,), jnp.int32),
            pltpu.VMEM((B_PER_W // 2, 2 * D_MODEL), jnp.float32),
            pltpu.SemaphoreType.DMA((4,)),
        ],
    )
    return run(x_entity0[:, 0], x_entity0[:, 1], embeddings).reshape(B, D_MODEL)
